# baseline (speedup 1.0000x reference)
import jax
import jax.numpy as jnp
from jax import lax
from jax.experimental import pallas as pl
from jax.experimental.pallas import tpu as pltpu
from jax.experimental.pallas import tpu_sc as plsc

B = 16384
F = 26
V = 100000
DC = 13

NC = 2
NS = 16
L = 16
NW = NC * NS
BPW = B // NW
NIDX = BPW * F
NCONT = BPW * DC
CHUNK = 128
NCHUNK = NIDX // CHUNK
VP = 100096


def _sc_body(tbl_ref, idx_ref, cont_ref, w_ref, out_ref,
             idx_v, vals_v, cont_v, w_v, acc_v, sem):
    wid = lax.axis_index("s") * NC + lax.axis_index("c")

    pltpu.sync_copy(idx_ref.at[pl.ds(wid * NIDX, NIDX)], idx_v)
    pltpu.sync_copy(cont_ref.at[pl.ds(wid * NCONT, NCONT)], cont_v)
    pltpu.sync_copy(w_ref, w_v)

    @pl.loop(0, NCHUNK)
    def fire(c):
        sl = pl.ds(c * CHUNK, CHUNK)
        pltpu.async_copy(tbl_ref.at[idx_v.at[sl]], vals_v.at[sl], sem)

    @pl.loop(0, NCHUNK)
    def drain(c):
        sl = pl.ds(c * CHUNK, CHUNK)
        pltpu.make_async_copy(tbl_ref.at[idx_v.at[sl]], vals_v.at[sl], sem).wait()

    w_cont = [w_v[pl.ds(L * c, L)] for c in range(DC)]
    w_emb = [w_v[pl.ds(L * (DC + f), L)] for f in range(F)]
    bias_v = w_v[pl.ds(L * (DC + F), L)]

    @pl.loop(0, BPW // L)
    def reduce_rows(j):
        acc = bias_v
        for c in range(DC):
            acc = acc + cont_v[pl.ds(c * BPW + j * L, L)] * w_cont[c]
        for f in range(F):
            acc = acc + vals_v[pl.ds(f * BPW + j * L, L)] * w_emb[f]
        acc_v[pl.ds(j * L, L)] = acc

    pltpu.sync_copy(acc_v, out_ref.at[pl.ds(wid * BPW, BPW)])


@jax.jit
def _run(tbl_flat, idx_bl, cont_bl, w_bc):
    mesh = plsc.VectorSubcoreMesh(core_axis_name="c", subcore_axis_name="s",
                                  num_cores=NC, num_subcores=NS)
    return pl.kernel(
        _sc_body,
        out_type=jax.ShapeDtypeStruct((B,), jnp.float32),
        mesh=mesh,
        compiler_params=pltpu.CompilerParams(needs_layout_passes=False,
                                             use_tc_tiling_on_sc=False),
        scratch_types=[
            pltpu.VMEM((NIDX,), jnp.int32),
            pltpu.VMEM((NIDX,), jnp.float32),
            pltpu.VMEM((NCONT,), jnp.float32),
            pltpu.VMEM(((DC + F + 1) * L, ), jnp.float32),
            pltpu.VMEM((BPW,), jnp.float32),
            pltpu.SemaphoreType.DMA,
        ],
    )(tbl_flat, idx_bl, cont_bl, w_bc)


def kernel(continuous, discrete, tables, W, b):
    zeros96 = jnp.zeros((VP - V,), jnp.float32)
    tbl_flat = jnp.concatenate(
        [x for f in range(F) for x in (tables[f, :, 0], zeros96)])
    idx = discrete.astype(jnp.int32) + jnp.arange(F, dtype=jnp.int32)[None, :] * VP
    idx_bl = idx.reshape(NW, BPW, F).transpose(0, 2, 1).reshape(NW * NIDX)
    cont_bl = continuous.reshape(NW, BPW, DC).transpose(0, 2, 1).reshape(NW * NCONT)
    w_row = jnp.concatenate([W.reshape(DC + F), b.reshape(1)])
    w_bc = jnp.repeat(w_row, L)
    out = _run(tbl_flat, idx_bl, cont_bl, w_bc)
    return out.reshape(B, 1)


# speedup vs baseline: 2.8613x; 2.8613x over previous
import jax
import jax.numpy as jnp
from jax import lax
from jax.experimental import pallas as pl
from jax.experimental.pallas import tpu as pltpu
from jax.experimental.pallas import tpu_sc as plsc

B = 16384
F = 26
V = 100000
DC = 13

NC = 2
NS = 16
L = 16
NW = NC * NS
BPW = B // NW
NIDX = BPW * F
NCONT = BPW * DC
CHUNK = 512
NCHUNK = NIDX // CHUNK
VP = 100096


def _sc_body(tbl_ref, idx_ref, cont_ref, w_ref, out_ref,
             idx_v, vals_v, cont_v, w_v, acc_v, sem):
    wid = lax.axis_index("s") * NC + lax.axis_index("c")

    pltpu.sync_copy(idx_ref.at[pl.ds(wid * NIDX, NIDX)], idx_v)

    @pl.loop(0, NCHUNK)
    def fire(c):
        sl = pl.ds(c * CHUNK, CHUNK)
        pltpu.async_copy(tbl_ref.at[idx_v.at[sl]], vals_v.at[sl], sem)

    pltpu.sync_copy(cont_ref.at[pl.ds(wid * NCONT, NCONT)], cont_v)
    pltpu.sync_copy(w_ref, w_v)

    @pl.loop(0, NCHUNK)
    def drain(c):
        sl = pl.ds(c * CHUNK, CHUNK)
        pltpu.make_async_copy(tbl_ref.at[idx_v.at[sl]], vals_v.at[sl], sem).wait()

    w_cont = [w_v[pl.ds(L * c, L)] for c in range(DC)]
    w_emb = [w_v[pl.ds(L * (DC + f), L)] for f in range(F)]
    bias_v = w_v[pl.ds(L * (DC + F), L)]

    @pl.loop(0, BPW // L)
    def reduce_rows(j):
        acc = bias_v
        for c in range(DC):
            acc = acc + cont_v[pl.ds(c * BPW + j * L, L)] * w_cont[c]
        for f in range(F):
            acc = acc + vals_v[pl.ds(f * BPW + j * L, L)] * w_emb[f]
        acc_v[pl.ds(j * L, L)] = acc

    pltpu.sync_copy(acc_v, out_ref.at[pl.ds(wid * BPW, BPW)])


@jax.jit
def _run(tbl_flat, idx_bl, cont_bl, w_bc):
    mesh = plsc.VectorSubcoreMesh(core_axis_name="c", subcore_axis_name="s",
                                  num_cores=NC, num_subcores=NS)
    return pl.kernel(
        _sc_body,
        out_type=jax.ShapeDtypeStruct((B,), jnp.float32),
        mesh=mesh,
        compiler_params=pltpu.CompilerParams(needs_layout_passes=False,
                                             use_tc_tiling_on_sc=False),
        scratch_types=[
            pltpu.VMEM((NIDX,), jnp.int32),
            pltpu.VMEM((NIDX,), jnp.float32),
            pltpu.VMEM((NCONT,), jnp.float32),
            pltpu.VMEM(((DC + F + 1) * L, ), jnp.float32),
            pltpu.VMEM((BPW,), jnp.float32),
            pltpu.SemaphoreType.DMA,
        ],
    )(tbl_flat, idx_bl, cont_bl, w_bc)


def kernel(continuous, discrete, tables, W, b):
    tbl_flat = jnp.pad(tables.reshape(F, V), ((0, 0), (0, VP - V))).reshape(F * VP)
    idx = discrete.astype(jnp.int32) + jnp.arange(F, dtype=jnp.int32)[None, :] * VP
    idx_bl = idx.reshape(NW, BPW, F).transpose(0, 2, 1).reshape(NW * NIDX)
    cont_bl = continuous.reshape(NW, BPW, DC).transpose(0, 2, 1).reshape(NW * NCONT)
    w_row = jnp.concatenate([W.reshape(DC + F), b.reshape(1)])
    w_bc = jnp.repeat(w_row, L)
    out = _run(tbl_flat, idx_bl, cont_bl, w_bc)
    return out.reshape(B, 1)
